# baseline (speedup 1.0000x reference)
import functools

import jax
import jax.numpy as jnp
from jax import lax
from jax.experimental import pallas as pl
from jax.experimental.pallas import tpu as pltpu
from jax.experimental.pallas import tpu_sc as plsc

ROWS = 8192
DIM = 2048
LANES = 16

NUM_CORES = 2
NUM_SUBCORES = 16
NUM_WORKERS = NUM_CORES * NUM_SUBCORES

ROWS_PER_WORKER = ROWS // NUM_WORKERS
CHUNK = 8
NCHUNKS = ROWS_PER_WORKER // CHUNK
NGROUPS = DIM // LANES


def _sc_permute(x_hbm, perm_hbm, out_hbm, perm_v, in_bufs, out_bufs,
                in_sems, out_sems):
  wid = lax.axis_index("s") * NUM_CORES + lax.axis_index("c")
  row0 = wid * ROWS_PER_WORKER

  pltpu.sync_copy(perm_hbm, perm_v)

  def copy_in(ch):
    b = ch % 2
    return pltpu.make_async_copy(
        x_hbm.at[pl.ds(row0 + ch * CHUNK, CHUNK)], in_bufs[b], in_sems[b])

  def copy_out(ch):
    b = ch % 2
    return pltpu.make_async_copy(
        out_bufs[b], out_hbm.at[pl.ds(row0 + ch * CHUNK, CHUNK)], out_sems[b])

  def gather_chunk(in_buf, out_buf):
    @plsc.parallel_loop(0, NGROUPS, unroll=8)
    def _(j):
      col0 = j * LANES
      idx = perm_v[pl.ds(col0, LANES)]
      for r in range(CHUNK):
        row = jnp.full((LANES,), r, dtype=jnp.int32)
        vals = plsc.load_gather(in_buf, [row, idx])
        out_buf[r, pl.ds(col0, LANES)] = vals

  copy_in(0).start()
  for ch in range(NCHUNKS):
    b = ch % 2
    copy_in(ch).wait()
    if ch + 1 < NCHUNKS:
      copy_in(ch + 1).start()
    if ch >= 2:
      copy_out(ch - 2).wait()
    gather_chunk(in_bufs[b], out_bufs[b])
    copy_out(ch).start()
  copy_out(NCHUNKS - 2).wait()
  copy_out(NCHUNKS - 1).wait()


@jax.jit
def _permute(x, perm):
  mesh = plsc.VectorSubcoreMesh(
      core_axis_name="c", subcore_axis_name="s", num_cores=NUM_CORES,
      num_subcores=NUM_SUBCORES)
  f = pl.kernel(
      _sc_permute,
      out_type=jax.ShapeDtypeStruct((ROWS, DIM), jnp.float32),
      mesh=mesh,
      compiler_params=pltpu.CompilerParams(
          use_tc_tiling_on_sc=True, needs_layout_passes=False),
      scratch_types=[
          pltpu.VMEM((DIM,), jnp.int32),
          [pltpu.VMEM((CHUNK, DIM), jnp.float32) for _ in range(2)],
          [pltpu.VMEM((CHUNK, DIM), jnp.float32) for _ in range(2)],
          [pltpu.SemaphoreType.DMA for _ in range(2)],
          [pltpu.SemaphoreType.DMA for _ in range(2)],
      ],
  )
  return f(x, perm)


def kernel(x, perm):
  return _permute(x, perm)


# speedup vs baseline: 1.1253x; 1.1253x over previous
import functools

import jax
import jax.numpy as jnp
from jax import lax
from jax.experimental import pallas as pl
from jax.experimental.pallas import tpu as pltpu
from jax.experimental.pallas import tpu_sc as plsc

ROWS = 8192
DIM = 2048
LANES = 16

NUM_CORES = 2
NUM_SUBCORES = 16
NUM_WORKERS = NUM_CORES * NUM_SUBCORES

ROWS_PER_WORKER = ROWS // NUM_WORKERS
CHUNK = 8
NBUF = 3
NCHUNKS = ROWS_PER_WORKER // CHUNK
NGROUPS = DIM // LANES


def _sc_permute(x_hbm, perm_hbm, out_hbm, perm_v, in_bufs, out_bufs,
                in_sems, out_sems):
  wid = lax.axis_index("s") * NUM_CORES + lax.axis_index("c")
  row0 = wid * ROWS_PER_WORKER

  pltpu.sync_copy(perm_hbm, perm_v)

  def copy_in(ch):
    b = ch % NBUF
    return pltpu.make_async_copy(
        x_hbm.at[pl.ds(row0 + ch * CHUNK, CHUNK)], in_bufs[b], in_sems[b])

  def copy_out(ch):
    b = ch % NBUF
    return pltpu.make_async_copy(
        out_bufs[b], out_hbm.at[pl.ds(row0 + ch * CHUNK, CHUNK)], out_sems[b])

  def gather_chunk(in_buf, out_buf):
    @plsc.parallel_loop(0, NGROUPS, unroll=4)
    def _(j):
      col0 = j * LANES
      idx = perm_v[pl.ds(col0, LANES)]
      for r in range(CHUNK):
        row = jnp.full((LANES,), r, dtype=jnp.int32)
        vals = plsc.load_gather(in_buf, [row, idx])
        out_buf[r, pl.ds(col0, LANES)] = vals

  copy_in(0).start()
  copy_in(1).start()
  for ch in range(NCHUNKS):
    b = ch % NBUF
    copy_in(ch).wait()
    if ch + 2 < NCHUNKS:
      copy_in(ch + 2).start()
    if ch >= NBUF:
      copy_out(ch - NBUF).wait()
    gather_chunk(in_bufs[b], out_bufs[b])
    copy_out(ch).start()
  for ch in range(NCHUNKS - NBUF, NCHUNKS):
    copy_out(ch).wait()


@jax.jit
def _permute(x, perm):
  mesh = plsc.VectorSubcoreMesh(
      core_axis_name="c", subcore_axis_name="s", num_cores=NUM_CORES,
      num_subcores=NUM_SUBCORES)
  f = pl.kernel(
      _sc_permute,
      out_type=jax.ShapeDtypeStruct((ROWS, DIM), jnp.float32),
      mesh=mesh,
      compiler_params=pltpu.CompilerParams(
          use_tc_tiling_on_sc=True, needs_layout_passes=False),
      scratch_types=[
          pltpu.VMEM((DIM,), jnp.int32),
          [pltpu.VMEM((CHUNK, DIM), jnp.float32) for _ in range(NBUF)],
          [pltpu.VMEM((CHUNK, DIM), jnp.float32) for _ in range(NBUF)],
          [pltpu.SemaphoreType.DMA for _ in range(NBUF)],
          [pltpu.SemaphoreType.DMA for _ in range(NBUF)],
      ],
  )
  return f(x, perm)


def kernel(x, perm):
  return _permute(x, perm)


# speedup vs baseline: 1.1380x; 1.0113x over previous
import jax
import jax.numpy as jnp
from jax import lax
from jax.experimental import pallas as pl
from jax.experimental.pallas import tpu as pltpu
from jax.experimental.pallas import tpu_sc as plsc

ROWS = 8192
DIM = 2048
LANES = 16

NUM_CORES = 2
NUM_SUBCORES = 16
NUM_WORKERS = NUM_CORES * NUM_SUBCORES

ROWS_PER_WORKER = ROWS // NUM_WORKERS
CHUNK = 8
NBUF_IN = 4
NBUF_OUT = 3
NCHUNKS = ROWS_PER_WORKER // CHUNK
NGROUPS = DIM // LANES


def _sc_permute(x_hbm, perm_hbm, out_hbm, perm_v, in_bufs, out_bufs,
                perm_sem, in_sems, out_sems):
  wid = lax.axis_index("s") * NUM_CORES + lax.axis_index("c")
  row0 = wid * ROWS_PER_WORKER

  perm_copy = pltpu.make_async_copy(perm_hbm, perm_v, perm_sem)
  perm_copy.start()

  def copy_in(ch):
    b = ch % NBUF_IN
    return pltpu.make_async_copy(
        x_hbm.at[pl.ds(row0 + ch * CHUNK, CHUNK)], in_bufs[b], in_sems[b])

  def copy_out(ch):
    b = ch % NBUF_OUT
    return pltpu.make_async_copy(
        out_bufs[b], out_hbm.at[pl.ds(row0 + ch * CHUNK, CHUNK)], out_sems[b])

  def gather_chunk(in_buf, out_buf):
    @plsc.parallel_loop(0, NGROUPS, unroll=4)
    def _(j):
      col0 = j * LANES
      idx = perm_v[pl.ds(col0, LANES)]
      for r in range(CHUNK):
        row = jnp.full((LANES,), r, dtype=jnp.int32)
        vals = plsc.load_gather(in_buf, [row, idx])
        out_buf[r, pl.ds(col0, LANES)] = vals

  for ch in range(NBUF_IN - 1):
    copy_in(ch).start()
  perm_copy.wait()
  for ch in range(NCHUNKS):
    bo = ch % NBUF_OUT
    copy_in(ch).wait()
    if ch + NBUF_IN - 1 < NCHUNKS:
      copy_in(ch + NBUF_IN - 1).start()
    if ch >= NBUF_OUT:
      copy_out(ch - NBUF_OUT).wait()
    gather_chunk(in_bufs[ch % NBUF_IN], out_bufs[bo])
    copy_out(ch).start()
  for ch in range(NCHUNKS - NBUF_OUT, NCHUNKS):
    copy_out(ch).wait()


@jax.jit
def _permute(x, perm):
  mesh = plsc.VectorSubcoreMesh(
      core_axis_name="c", subcore_axis_name="s", num_cores=NUM_CORES,
      num_subcores=NUM_SUBCORES)
  f = pl.kernel(
      _sc_permute,
      out_type=jax.ShapeDtypeStruct((ROWS, DIM), jnp.float32),
      mesh=mesh,
      compiler_params=pltpu.CompilerParams(
          use_tc_tiling_on_sc=True, needs_layout_passes=False),
      scratch_types=[
          pltpu.VMEM((DIM,), jnp.int32),
          [pltpu.VMEM((CHUNK, DIM), jnp.float32) for _ in range(NBUF_IN)],
          [pltpu.VMEM((CHUNK, DIM), jnp.float32) for _ in range(NBUF_OUT)],
          pltpu.SemaphoreType.DMA,
          [pltpu.SemaphoreType.DMA for _ in range(NBUF_IN)],
          [pltpu.SemaphoreType.DMA for _ in range(NBUF_OUT)],
      ],
  )
  return f(x, perm)


def kernel(x, perm):
  return _permute(x, perm)
